# baseline (speedup 1.0000x reference)
import jax
import jax.numpy as jnp
from jax import lax
from jax.experimental import pallas as pl
from jax.experimental.pallas import tpu as pltpu
from jax.experimental.pallas import tpu_sc as plsc

_BB = 16
_TPAD = 1024
_RPW = 8


def _sc_gather_body(times_hbm, values_hbm, t_hbm, out_hbm, tvm, tv, rows_v, col_v):
    cid = lax.axis_index("c")
    sid = lax.axis_index("s")
    wid = sid * 2 + cid
    B, K = values_hbm.shape

    @pl.when(wid < B // _RPW)
    def _():
        base = wid * _RPW
        pltpu.sync_copy(times_hbm, tvm)
        pltpu.sync_copy(t_hbm, tv)
        pltpu.sync_copy(values_hbm.at[pl.ds(base, _RPW), :], rows_v)
        tvec = tv[...]

        def body(i, cnt):
            chunk = tvm[pl.ds(i * 16, 16)]
            return cnt + jnp.where(chunk <= tvec, 1, 0)

        cnt_vec = lax.fori_loop(0, _TPAD // 16, body,
                                jnp.zeros((16,), jnp.int32))
        count = jnp.sum(cnt_vec)
        idx = jnp.clip(count - 1, 0, K - 1)
        ridx = jnp.bitwise_and(lax.iota(jnp.int32, 16), _RPW - 1)
        cidx = jnp.full((16,), idx, jnp.int32)
        col_v[...] = plsc.load_gather(rows_v, [ridx, cidx])
        pltpu.sync_copy(col_v.at[pl.ds(0, _RPW)], out_hbm.at[pl.ds(base, _RPW)])


def _tc_scale_body(times_ref, vals_ref, t_ref, arr_ref, out_ref):
    tt = t_ref[0, 0]
    Kp1 = times_ref.shape[-1]
    flag = jnp.where(
        jnp.logical_and(times_ref[0, 0] <= tt, tt < times_ref[0, Kp1 - 1]),
        jnp.float32(1.0), jnp.float32(0.0),
    )
    vals = vals_ref[...] * flag
    out_ref[...] = vals[:, :, None] * arr_ref[...][None]


@jax.jit
def kernel(times, values, array, t):
    B, K = values.shape
    N = array.shape[0]

    times_pad = jnp.pad(times, (0, _TPAD - (K + 1)), constant_values=jnp.inf)
    t_vec = jnp.full((16,), t, jnp.float32)

    mesh = plsc.VectorSubcoreMesh(core_axis_name="c", subcore_axis_name="s")
    envelope = pl.kernel(
        _sc_gather_body,
        out_type=jax.ShapeDtypeStruct((B,), jnp.float32),
        mesh=mesh,
        compiler_params=pltpu.CompilerParams(needs_layout_passes=False),
        scratch_types=[
            pltpu.VMEM((_TPAD,), jnp.float32),
            pltpu.VMEM((16,), jnp.float32),
            pltpu.VMEM((_RPW, K), jnp.float32),
            pltpu.VMEM((16,), jnp.float32),
        ],
    )(times_pad, values, t_vec)

    return pl.pallas_call(
        _tc_scale_body,
        grid=(B // _BB,),
        in_specs=[
            pl.BlockSpec((1, K + 1), lambda b: (0, 0)),
            pl.BlockSpec((_BB, 1), lambda b: (b, 0)),
            pl.BlockSpec((1, 1), lambda b: (0, 0)),
            pl.BlockSpec((N, N), lambda b: (0, 0)),
        ],
        out_specs=pl.BlockSpec((_BB, N, N), lambda b: (b, 0, 0)),
        out_shape=jax.ShapeDtypeStruct((B, N, N), jnp.float32),
        compiler_params=pltpu.CompilerParams(
            dimension_semantics=(pltpu.GridDimensionSemantics.PARALLEL,)
        ),
    )(times.reshape(1, K + 1), envelope.reshape(B, 1), t.reshape(1, 1), array)


# speedup vs baseline: 2.9447x; 2.9447x over previous
import jax
import jax.numpy as jnp
from jax.experimental import pallas as pl
from jax.experimental.pallas import tpu as pltpu

_BB = 16
_NR = 128


def _pwc_body(times_ref, vals_ref, t_ref, arr_ref, out_ref):
    tt = t_ref[0, 0]
    K = vals_ref.shape[-1]
    t_lo = times_ref[0, :K]
    t_hi = times_ref[0, 1:]
    mask = (t_lo <= tt) & (tt < t_hi)
    vals = jnp.where(mask[None, :], vals_ref[...], 0.0).sum(axis=1)
    out_ref[...] = vals[:, None, None] * arr_ref[...][None]


@jax.jit
def kernel(times, values, array, t):
    B, K = values.shape
    N = array.shape[0]

    return pl.pallas_call(
        _pwc_body,
        grid=(B // _BB, N // _NR),
        in_specs=[
            pl.BlockSpec((1, K + 1), lambda b, r: (0, 0)),
            pl.BlockSpec((_BB, K), lambda b, r: (b, 0)),
            pl.BlockSpec((1, 1), lambda b, r: (0, 0)),
            pl.BlockSpec((_NR, N), lambda b, r: (r, 0)),
        ],
        out_specs=pl.BlockSpec((_BB, _NR, N), lambda b, r: (b, r, 0)),
        out_shape=jax.ShapeDtypeStruct((B, N, N), jnp.float32),
        compiler_params=pltpu.CompilerParams(
            dimension_semantics=(
                pltpu.GridDimensionSemantics.PARALLEL,
                pltpu.GridDimensionSemantics.PARALLEL,
            )
        ),
    )(times.reshape(1, K + 1), values, t.reshape(1, 1), array)


# speedup vs baseline: 3.6268x; 1.2316x over previous
import jax
import jax.numpy as jnp
from jax.experimental import pallas as pl
from jax.experimental.pallas import tpu as pltpu

_BB = 16


def _pwc_body(times_ref, vals_ref, t_ref, arr_ref, out_ref):
    tt = t_ref[0, 0]
    K = vals_ref.shape[-1]
    t_lo = times_ref[0, :K]
    t_hi = times_ref[0, 1:]
    mask = (t_lo <= tt) & (tt < t_hi)
    vals = jnp.where(mask[None, :], vals_ref[...], 0.0).sum(axis=1)
    out_ref[...] = vals[:, None, None] * arr_ref[...][None]


@jax.jit
def kernel(times, values, array, t):
    B, K = values.shape
    N = array.shape[0]

    return pl.pallas_call(
        _pwc_body,
        grid=(B // _BB,),
        in_specs=[
            pl.BlockSpec((1, K + 1), lambda b: (0, 0)),
            pl.BlockSpec((_BB, K), lambda b: (b, 0)),
            pl.BlockSpec((1, 1), lambda b: (0, 0)),
            pl.BlockSpec((N, N), lambda b: (0, 0)),
        ],
        out_specs=pl.BlockSpec((_BB, N, N), lambda b: (b, 0, 0)),
        out_shape=jax.ShapeDtypeStruct((B, N, N), jnp.float32),
        compiler_params=pltpu.CompilerParams(
            dimension_semantics=(pltpu.GridDimensionSemantics.PARALLEL,)
        ),
    )(times.reshape(1, K + 1), values, t.reshape(1, 1), array)
